# baseline (speedup 1.0000x reference)
import functools

import jax
import jax.numpy as jnp
from jax import lax
from jax.experimental import pallas as pl
from jax.experimental.pallas import tpu as pltpu
from jax.experimental.pallas import tpu_sc as plsc

N = 10000
NP = 10016
E = 160000
EP = 163840
D = 256
DH = 128
D2 = 128
RB = 2048
TPR = 632
C1 = 80
C2 = 40
NCH = EP // 128


def _mesh():
    return plsc.VectorSubcoreMesh(core_axis_name="c", subcore_axis_name="s")


EB = 16384


def _count_hi_lo(dst_ref):
    c = jnp.zeros((80, 128), jnp.float32)
    for r in range(EP // EB):
        d = dst_ref[r]
        hi = jax.lax.shift_right_logical(d, 7)
        lo = jax.lax.bitwise_and(d, 127)
        a = (jax.lax.broadcasted_iota(jnp.int32, (80, EB), 0) == hi[None, :]
             ).astype(jnp.bfloat16)
        b = (jax.lax.broadcasted_iota(jnp.int32, (128, EB), 0) == lo[None, :]
             ).astype(jnp.bfloat16)
        c = c + jax.lax.dot_general(a, b, (((1,), (1,)), ((), ())),
                                    preferred_element_type=jnp.float32)
    return c


def _dinv3(c_blk):
    return lax.rsqrt(c_blk + 1.0)[:, :, None]


def _agg_ring(tab_hbm, sd_hbm, zeros_hbm, out_hbm, r0, cid, base, acc, idxv,
              gbuf, gsems, ssems, isems, zsem, nchunks):
    last = r0 == 15 * TPR

    @pl.when(~last)
    def _():
        pltpu.async_copy(zeros_hbm.at[pl.ds(0, TPR)], acc.at[pl.ds(r0, TPR)],
                         zsem)

    @pl.when(last)
    def _():
        pltpu.async_copy(zeros_hbm.at[pl.ds(0, 536)],
                         acc.at[pl.ds(15 * TPR, 536)], zsem)

    for j in range(4):
        pltpu.async_copy(sd_hbm.at[base + j], idxv.at[j], isems[j])
    for b in range(3):
        pltpu.make_async_copy(sd_hbm.at[base], idxv.at[b], isems[b]).wait()
        pltpu.async_copy(tab_hbm.at[idxv.at[b, 0]], gbuf.at[b], gsems[b])
    @pl.when(~last)
    def _():
        pltpu.make_async_copy(zeros_hbm.at[pl.ds(0, TPR)],
                              acc.at[pl.ds(r0, TPR)], zsem).wait()

    @pl.when(last)
    def _():
        pltpu.make_async_copy(zeros_hbm.at[pl.ds(0, 536)],
                              acc.at[pl.ds(15 * TPR, 536)], zsem).wait()

    plsc.subcore_barrier()

    def step(c, b12, dyn):
        sb = b12 % 3
        pbuf = (b12 + 2) % 3
        sj = b12 % 4
        jr = (b12 + 3) % 4
        jg = (b12 + 2) % 4
        pltpu.make_async_copy(tab_hbm.at[idxv.at[0, 0]], gbuf.at[sb],
                              gsems[sb]).wait()
        pltpu.async_copy(gbuf.at[sb], acc.at[idxv.at[sj, 1]], ssems[sb],
                         add=True)

        def wait_prev_scatter():
            pltpu.make_async_copy(gbuf.at[pbuf], acc.at[idxv.at[0, 1]],
                                  ssems[pbuf]).wait()

        def reload_idx():
            pltpu.async_copy(sd_hbm.at[base + c + 3], idxv.at[jr], isems[jr])

        def refill_gather():
            pltpu.make_async_copy(sd_hbm.at[base], idxv.at[jg],
                                  isems[jg]).wait()
            pltpu.async_copy(tab_hbm.at[idxv.at[jg, 0]], gbuf.at[pbuf],
                             gsems[pbuf])

        if dyn:
            pl.when(c >= 1)(wait_prev_scatter)
            pl.when((c >= 1) & (c + 3 < nchunks))(reload_idx)
            pl.when((c >= 1) & (c + 2 < nchunks))(refill_gather)
        else:
            if c >= 1:
                wait_prev_scatter()
            if c >= 1 and c + 3 < nchunks:
                reload_idx()
            if c >= 1 and c + 2 < nchunks:
                refill_gather()

    ngrp = nchunks // 12

    def group(g, carry):
        for b12 in range(12):
            step(g * 12 + b12, b12, True)
        return carry

    lax.fori_loop(0, ngrp, group, 0)
    for k in range(nchunks % 12):
        step(ngrp * 12 + k, k, False)
    pltpu.make_async_copy(gbuf.at[(nchunks - 1) % 3], acc.at[idxv.at[0, 1]],
                          ssems[(nchunks - 1) % 3]).wait()
    plsc.subcore_barrier()

    @pl.when(~last)
    def _():
        pltpu.sync_copy(acc.at[pl.ds(r0, TPR)],
                        out_hbm.at[pl.ds(cid * NP + r0, TPR)])

    @pl.when(last)
    def _():
        pltpu.sync_copy(acc.at[pl.ds(15 * TPR, 536)],
                        out_hbm.at[pl.ds(cid * NP + 15 * TPR, 536)])


_AGG_SCRATCH = [
    pltpu.VMEM((4, 2, 128), jnp.int32),
    pltpu.VMEM((3, 128, 128), jnp.float32),
] + [pltpu.SemaphoreType.DMA] * 11


@functools.partial(
    pl.kernel,
    out_type=jax.ShapeDtypeStruct((2 * NP, DH), jnp.float32),
    mesh=_mesh(),
    scratch_types=[pltpu.VMEM_SHARED((NP, DH), jnp.float32)] + _AGG_SCRATCH,
)
def _agg1_kernel(tab_hbm, sd_hbm, zeros_hbm, out_hbm,
                 acc, idxv, gbuf, g0, g1, g2, s0, s1, s2,
                 i0, i1, i2, i3, z0):
    c = lax.axis_index("c")
    s = lax.axis_index("s")
    _agg_ring(tab_hbm, sd_hbm, zeros_hbm, out_hbm, s * TPR, c,
              c * NCH + s * C1, acc, idxv, gbuf,
              [g0, g1, g2], [s0, s1, s2], [i0, i1, i2, i3], z0, C1)


@functools.partial(
    pl.kernel,
    out_type=jax.ShapeDtypeStruct((2 * NP, D2), jnp.float32),
    mesh=_mesh(),
    scratch_types=[pltpu.VMEM_SHARED((NP, D2), jnp.float32)] + _AGG_SCRATCH,
)
def _agg2_kernel(tab_hbm, sd_hbm, zeros_hbm, out_hbm,
                 acc, idxv, gbuf, g0, g1, g2, s0, s1, s2,
                 i0, i1, i2, i3, z0):
    c = lax.axis_index("c")
    s = lax.axis_index("s")
    _agg_ring(tab_hbm, sd_hbm, zeros_hbm, out_hbm, s * TPR, c,
              (c * 16 + s) * C2, acc, idxv, gbuf,
              [g0, g1, g2], [s0, s1, s2], [i0, i1, i2, i3], z0, C2)


def _k1_body(dst_ref, x_ref, w_ref, cout_ref, out_ref, cscr):
    i = pl.program_id(0)

    @pl.when(i == 0)
    def _():
        c = _count_hi_lo(dst_ref)
        cscr[...] = c
        cout_ref[...] = c

    dinv3 = _dinv3(cscr[pl.ds(i * 16, 16)])
    y = jnp.dot(x_ref[...], w_ref[...], preferred_element_type=jnp.float32)
    y = (y.reshape(16, 128, D) * dinv3).reshape(RB, D)
    out_ref[0] = y[:, :DH]
    out_ref[1] = y[:, DH:]


def _k1(dstE, x, W1):
    return pl.pallas_call(
        _k1_body,
        grid=((NP + RB - 1) // RB,),
        in_specs=[
            pl.BlockSpec((EP // EB, EB), lambda i: (0, 0)),
            pl.BlockSpec((RB, D), lambda i: (i, 0)),
            pl.BlockSpec((D, D), lambda i: (0, 0)),
        ],
        out_specs=[
            pl.BlockSpec((80, 128), lambda i: (0, 0)),
            pl.BlockSpec((2, RB, DH), lambda i: (0, i, 0)),
        ],
        out_shape=[
            jax.ShapeDtypeStruct((80, 128), jnp.float32),
            jax.ShapeDtypeStruct((2, NP, DH), jnp.float32),
        ],
        scratch_shapes=[pltpu.VMEM((80, 128), jnp.float32)],
    )(dstE, x, W1)


def _k2_body(c_ref, agg_ref, xs_ref, b1_ref, w2_ref, x1_ref, xw2_ref):
    i = pl.program_id(0)
    dinv3 = _dinv3(c_ref[pl.ds(i * 16, 16)])
    h = jnp.concatenate(
        [agg_ref[0] + xs_ref[0], agg_ref[1] + xs_ref[1]], axis=1)
    h = (h.reshape(16, 128, D) * dinv3).reshape(RB, D)
    x1 = jnp.maximum(h + b1_ref[...], 0.0)
    x1_ref[...] = x1
    y2 = jnp.dot(x1, w2_ref[...], preferred_element_type=jnp.float32)
    xw2_ref[...] = (y2.reshape(16, 128, D2) * dinv3).reshape(RB, D2)


def _k2(degC, aggR, xsR, b1, W2p):
    return pl.pallas_call(
        _k2_body,
        grid=((NP + RB - 1) // RB,),
        in_specs=[
            pl.BlockSpec((80, 128), lambda i: (0, 0)),
            pl.BlockSpec((2, RB, DH), lambda i: (0, i, 0)),
            pl.BlockSpec((2, RB, DH), lambda i: (0, i, 0)),
            pl.BlockSpec((1, D), lambda i: (0, 0)),
            pl.BlockSpec((D, D2), lambda i: (0, 0)),
        ],
        out_specs=[
            pl.BlockSpec((RB, D), lambda i: (i, 0)),
            pl.BlockSpec((RB, D2), lambda i: (i, 0)),
        ],
        out_shape=[
            jax.ShapeDtypeStruct((N, D), jnp.float32),
            jax.ShapeDtypeStruct((NP, D2), jnp.float32),
        ],
    )(degC, aggR, xsR, b1, W2p)


def _k3_body(c_ref, agg_ref, xw2_ref, b2_ref, out_ref):
    i = pl.program_id(0)
    dinv3 = _dinv3(c_ref[pl.ds(i * 16, 16)])
    h = agg_ref[0] + agg_ref[1] + xw2_ref[...]
    h = (h.reshape(16, 128, D2) * dinv3).reshape(RB, D2)
    out_ref[...] = h[:, :40] + b2_ref[...]


def _k3(degC, agg2R, xw2, b2p):
    return pl.pallas_call(
        _k3_body,
        grid=((NP + RB - 1) // RB,),
        in_specs=[
            pl.BlockSpec((80, 128), lambda i: (0, 0)),
            pl.BlockSpec((2, RB, D2), lambda i: (0, i, 0)),
            pl.BlockSpec((RB, D2), lambda i: (i, 0)),
            pl.BlockSpec((1, 40), lambda i: (0, 0)),
        ],
        out_specs=pl.BlockSpec((RB, 40), lambda i: (i, 0)),
        out_shape=jax.ShapeDtypeStruct((N, 40), jnp.float32),
    )(degC, agg2R, xw2, b2p)


@jax.jit
def _run(x, edge_index, W1, b1, W2, b2):
    ei = edge_index.astype(jnp.int32)
    pad_idx = N + (jnp.arange(EP - E, dtype=jnp.int32) % (NP - N))
    src = jnp.concatenate([ei[0], pad_idx])
    dst = jnp.concatenate([ei[1], pad_idx])
    src2d = src.reshape(NCH, 128)
    dst2d = dst.reshape(NCH, 128)
    sd = jnp.stack([src2d, dst2d], axis=1)
    sd1 = jnp.concatenate(
        [sd, jnp.stack([src2d + NP, dst2d], axis=1)], axis=0)
    xp = jnp.pad(x, ((0, NP - N), (0, 0)))
    W2p = jnp.pad(W2, ((0, 0), (0, D2 - W2.shape[1])))
    b2p = b2.reshape(1, 40)
    zeros128 = jnp.zeros((TPR, DH), jnp.float32)

    degC, xs1 = _k1(dst.reshape(EP // EB, EB), xp, W1)
    agg1 = _agg1_kernel(xs1.reshape(2 * NP, DH), sd1, zeros128)
    x1, xw2 = _k2(degC, agg1.reshape(2, NP, DH), xs1,
                  b1.reshape(1, D), W2p)
    agg2 = _agg2_kernel(xw2, sd, zeros128)
    x2 = _k3(degC, agg2.reshape(2, NP, D2), xw2, b2p)
    return x1, x2


def kernel(x, edge_index, W1, b1, W2, b2):
    return _run(x, edge_index, W1, b1, W2, b2)


# speedup vs baseline: 1.0449x; 1.0449x over previous
import functools

import jax
import jax.numpy as jnp
from jax import lax
from jax.experimental import pallas as pl
from jax.experimental.pallas import tpu as pltpu
from jax.experimental.pallas import tpu_sc as plsc

N = 10000
NP = 10016
E = 160000
EP = 163840
D = 256
DH = 128
D2 = 128
RB = 2048
TPR = 632
C1 = 80
C2 = 40
NCH = EP // 128


def _mesh():
    return plsc.VectorSubcoreMesh(core_axis_name="c", subcore_axis_name="s")


EB = 16384


def _count_hi_lo(dst_ref):
    c = jnp.zeros((80, 128), jnp.float32)
    for r in range(EP // EB):
        d = dst_ref[r]
        hi = jax.lax.shift_right_logical(d, 7)
        lo = jax.lax.bitwise_and(d, 127)
        a = (jax.lax.broadcasted_iota(jnp.int32, (80, EB), 0) == hi[None, :]
             ).astype(jnp.bfloat16)
        b = (jax.lax.broadcasted_iota(jnp.int32, (128, EB), 0) == lo[None, :]
             ).astype(jnp.bfloat16)
        c = c + jax.lax.dot_general(a, b, (((1,), (1,)), ((), ())),
                                    preferred_element_type=jnp.float32)
    return c


def _dinv3(c_blk):
    return lax.rsqrt(c_blk + 1.0)[:, :, None]


def _agg_ring(tab_hbm, sd_hbm, zeros_hbm, out_hbm, r0, cid, base, acc, idxv,
              gbuf, gsems, ssems, isems, zsem, nchunks):
    last = r0 == 15 * TPR

    @pl.when(~last)
    def _():
        pltpu.async_copy(zeros_hbm.at[pl.ds(0, TPR)], acc.at[pl.ds(r0, TPR)],
                         zsem)

    @pl.when(last)
    def _():
        pltpu.async_copy(zeros_hbm.at[pl.ds(0, 536)],
                         acc.at[pl.ds(15 * TPR, 536)], zsem)

    for j in range(4):
        pltpu.async_copy(sd_hbm.at[base + j], idxv.at[j], isems[j])
    for b in range(3):
        pltpu.make_async_copy(sd_hbm.at[base], idxv.at[b], isems[b]).wait()
        pltpu.async_copy(tab_hbm.at[idxv.at[b, 0]], gbuf.at[b], gsems[b])
    @pl.when(~last)
    def _():
        pltpu.make_async_copy(zeros_hbm.at[pl.ds(0, TPR)],
                              acc.at[pl.ds(r0, TPR)], zsem).wait()

    @pl.when(last)
    def _():
        pltpu.make_async_copy(zeros_hbm.at[pl.ds(0, 536)],
                              acc.at[pl.ds(15 * TPR, 536)], zsem).wait()

    plsc.subcore_barrier()

    def step(c, b12, dyn):
        sb = b12 % 3
        pbuf = (b12 + 2) % 3
        sj = b12 % 4
        jr = (b12 + 3) % 4
        jg = (b12 + 2) % 4
        pltpu.make_async_copy(tab_hbm.at[idxv.at[0, 0]], gbuf.at[sb],
                              gsems[sb]).wait()
        pltpu.async_copy(gbuf.at[sb], acc.at[idxv.at[sj, 1]], ssems[sb],
                         add=True)

        def wait_prev_scatter():
            pltpu.make_async_copy(gbuf.at[pbuf], acc.at[idxv.at[0, 1]],
                                  ssems[pbuf]).wait()

        def reload_idx():
            pltpu.async_copy(sd_hbm.at[base + c + 3], idxv.at[jr], isems[jr])

        def refill_gather():
            pltpu.make_async_copy(sd_hbm.at[base], idxv.at[jg],
                                  isems[jg]).wait()
            pltpu.async_copy(tab_hbm.at[idxv.at[jg, 0]], gbuf.at[pbuf],
                             gsems[pbuf])

        if dyn:
            pl.when(c >= 1)(wait_prev_scatter)
            pl.when((c >= 1) & (c + 3 < nchunks))(reload_idx)
            pl.when((c >= 1) & (c + 2 < nchunks))(refill_gather)
        else:
            if c >= 1:
                wait_prev_scatter()
            if c >= 1 and c + 3 < nchunks:
                reload_idx()
            if c >= 1 and c + 2 < nchunks:
                refill_gather()

    ngrp = nchunks // 12

    def group(g, carry):
        for b12 in range(12):
            step(g * 12 + b12, b12, True)
        return carry

    lax.fori_loop(0, ngrp, group, 0)
    for k in range(nchunks % 12):
        step(ngrp * 12 + k, k, False)
    pltpu.make_async_copy(gbuf.at[(nchunks - 1) % 3], acc.at[idxv.at[0, 1]],
                          ssems[(nchunks - 1) % 3]).wait()
    plsc.subcore_barrier()

    @pl.when(~last)
    def _():
        pltpu.sync_copy(acc.at[pl.ds(r0, TPR)],
                        out_hbm.at[pl.ds(cid * NP + r0, TPR)])

    @pl.when(last)
    def _():
        pltpu.sync_copy(acc.at[pl.ds(15 * TPR, 536)],
                        out_hbm.at[pl.ds(cid * NP + 15 * TPR, 536)])


_AGG_SCRATCH = [
    pltpu.VMEM((4, 2, 128), jnp.int32),
    pltpu.VMEM((3, 128, 128), jnp.float32),
] + [pltpu.SemaphoreType.DMA] * 11


@functools.partial(
    pl.kernel,
    out_type=jax.ShapeDtypeStruct((2 * NP, DH), jnp.float32),
    mesh=_mesh(),
    scratch_types=[pltpu.VMEM_SHARED((NP, DH), jnp.float32)] + _AGG_SCRATCH,
)
def _agg1_kernel(tab_hbm, sd_hbm, zeros_hbm, out_hbm,
                 acc, idxv, gbuf, g0, g1, g2, s0, s1, s2,
                 i0, i1, i2, i3, z0):
    c = lax.axis_index("c")
    s = lax.axis_index("s")
    _agg_ring(tab_hbm, sd_hbm, zeros_hbm, out_hbm, s * TPR, c,
              c * NCH + s * C1, acc, idxv, gbuf,
              [g0, g1, g2], [s0, s1, s2], [i0, i1, i2, i3], z0, C1)


@functools.partial(
    pl.kernel,
    out_type=jax.ShapeDtypeStruct((2 * NP, D2), jnp.float32),
    mesh=_mesh(),
    scratch_types=[pltpu.VMEM_SHARED((NP, D2), jnp.float32)] + _AGG_SCRATCH,
)
def _agg2_kernel(tab_hbm, sd_hbm, zeros_hbm, out_hbm,
                 acc, idxv, gbuf, g0, g1, g2, s0, s1, s2,
                 i0, i1, i2, i3, z0):
    c = lax.axis_index("c")
    s = lax.axis_index("s")
    _agg_ring(tab_hbm, sd_hbm, zeros_hbm, out_hbm, s * TPR, c,
              (c * 16 + s) * C2, acc, idxv, gbuf,
              [g0, g1, g2], [s0, s1, s2], [i0, i1, i2, i3], z0, C2)


def _k1_body(dst_ref, x_ref, w_ref, cout_ref, out_ref, cscr):
    i = pl.program_id(0)

    @pl.when(i == 0)
    def _():
        c = _count_hi_lo(dst_ref)
        cscr[...] = c
        cout_ref[...] = c

    dinv3 = _dinv3(cscr[pl.ds(i * 16, 16)])
    y = jnp.dot(x_ref[...], w_ref[...], preferred_element_type=jnp.float32)
    y = (y.reshape(16, 128, D) * dinv3).reshape(RB, D)
    out_ref[0] = y[:, :DH]
    out_ref[1] = y[:, DH:]


def _k1(dstE, x, W1):
    return pl.pallas_call(
        _k1_body,
        grid=((NP + RB - 1) // RB,),
        in_specs=[
            pl.BlockSpec((EP // EB, EB), lambda i: (0, 0)),
            pl.BlockSpec((RB, D), lambda i: (i, 0)),
            pl.BlockSpec((D, D), lambda i: (0, 0)),
        ],
        out_specs=[
            pl.BlockSpec((80, 128), lambda i: (0, 0)),
            pl.BlockSpec((2, RB, DH), lambda i: (0, i, 0)),
        ],
        out_shape=[
            jax.ShapeDtypeStruct((80, 128), jnp.float32),
            jax.ShapeDtypeStruct((2, NP, DH), jnp.float32),
        ],
        scratch_shapes=[pltpu.VMEM((80, 128), jnp.float32)],
    )(dstE, x, W1)


def _k2_body(c_ref, agg_ref, xs_ref, b1_ref, w2_ref, x1_ref, xw2_ref):
    i = pl.program_id(0)
    dinv3 = _dinv3(c_ref[pl.ds(i * 16, 16)])
    h = jnp.concatenate(
        [agg_ref[0] + xs_ref[0], agg_ref[1] + xs_ref[1]], axis=1)
    h = (h.reshape(16, 128, D) * dinv3).reshape(RB, D)
    x1 = jnp.maximum(h + b1_ref[...], 0.0)
    x1_ref[...] = x1
    y2 = jnp.dot(x1, w2_ref[...], preferred_element_type=jnp.float32)
    xw2_ref[...] = (y2.reshape(16, 128, D2) * dinv3).reshape(RB, D2)


def _k2(degC, aggR, xsR, b1, W2p):
    return pl.pallas_call(
        _k2_body,
        grid=((NP + RB - 1) // RB,),
        in_specs=[
            pl.BlockSpec((80, 128), lambda i: (0, 0)),
            pl.BlockSpec((2, RB, DH), lambda i: (0, i, 0)),
            pl.BlockSpec((2, RB, DH), lambda i: (0, i, 0)),
            pl.BlockSpec((1, D), lambda i: (0, 0)),
            pl.BlockSpec((D, D2), lambda i: (0, 0)),
        ],
        out_specs=[
            pl.BlockSpec((RB, D), lambda i: (i, 0)),
            pl.BlockSpec((RB, D2), lambda i: (i, 0)),
        ],
        out_shape=[
            jax.ShapeDtypeStruct((N, D), jnp.float32),
            jax.ShapeDtypeStruct((NP, D2), jnp.float32),
        ],
    )(degC, aggR, xsR, b1, W2p)


def _k3_body(c_ref, agg_ref, xw2_ref, b2_ref, out_ref):
    i = pl.program_id(0)
    dinv3 = _dinv3(c_ref[pl.ds(i * 16, 16)])
    h = agg_ref[0] + agg_ref[1] + xw2_ref[...]
    h = (h.reshape(16, 128, D2) * dinv3).reshape(RB, D2)
    out_ref[...] = h[:, :40] + b2_ref[...]


def _k3(degC, agg2R, xw2, b2p):
    return pl.pallas_call(
        _k3_body,
        grid=((NP + RB - 1) // RB,),
        in_specs=[
            pl.BlockSpec((80, 128), lambda i: (0, 0)),
            pl.BlockSpec((2, RB, D2), lambda i: (0, i, 0)),
            pl.BlockSpec((RB, D2), lambda i: (i, 0)),
            pl.BlockSpec((1, 40), lambda i: (0, 0)),
        ],
        out_specs=pl.BlockSpec((RB, 40), lambda i: (i, 0)),
        out_shape=jax.ShapeDtypeStruct((N, 40), jnp.float32),
    )(degC, agg2R, xw2, b2p)


@jax.jit
def _run(x, edge_index, W1, b1, W2, b2):
    ei = edge_index.astype(jnp.int32)
    pad_idx = N + (jnp.arange(EP - E, dtype=jnp.int32) % (NP - N))
    src = jnp.concatenate([ei[0], pad_idx])
    dst = jnp.concatenate([ei[1], pad_idx])
    src2d = src.reshape(NCH, 128)
    dst2d = dst.reshape(NCH, 128)
    sd = jnp.stack([src2d, dst2d], axis=1)
    sd1 = jnp.concatenate(
        [sd, jnp.stack([src2d + NP, dst2d], axis=1)], axis=0)
    W2p = jnp.pad(W2, ((0, 0), (0, D2 - W2.shape[1])))
    b2p = b2.reshape(1, 40)
    zeros128 = jnp.zeros((TPR, DH), jnp.float32)

    degC, xs1 = _k1(dst.reshape(EP // EB, EB), x, W1)
    agg1 = _agg1_kernel(xs1.reshape(2 * NP, DH), sd1, zeros128)
    x1, xw2 = _k2(degC, agg1.reshape(2, NP, DH), xs1,
                  b1.reshape(1, D), W2p)
    agg2 = _agg2_kernel(xw2, sd, zeros128)
    x2 = _k3(degC, agg2.reshape(2, NP, D2), xw2, b2p)
    return x1, x2


def kernel(x, edge_index, W1, b1, W2, b2):
    return _run(x, edge_index, W1, b1, W2, b2)
